# baseline (speedup 1.0000x reference)
import functools

import jax
import jax.numpy as jnp
from jax.experimental import pallas as pl
from jax.experimental.pallas import tpu as pltpu
from jax.experimental.pallas import tpu_sc as plsc

_TOPK = 2
_CAPF = 1.1
_SC_W = 128
_DQ = 256


def _router_kernel(x_ref, wg_ref, bg_ref, wn_ref, bn_ref, nz_ref,
                   g0_ref, g1_ref, s0_ref, s1_ref, loss_ref,
                   *, cap, cap_pad, n_exp):
    x = x_ref[...].astype(jnp.bfloat16)
    lg = jnp.dot(x, wg_ref[...].astype(jnp.bfloat16),
                 preferred_element_type=jnp.float32) + bg_ref[...]
    ln = jnp.dot(x, wn_ref[...].astype(jnp.bfloat16),
                 preferred_element_type=jnp.float32) + bn_ref[...]
    sp = jnp.maximum(ln, 0.0) + jnp.log1p(jnp.exp(-jnp.abs(ln)))
    noisy = lg + nz_ref[...] * sp
    n_tok = noisy.shape[0]

    eidx = jax.lax.broadcasted_iota(jnp.int32, noisy.shape, 1)
    m1 = jnp.max(noisy, axis=1, keepdims=True)
    i1 = jnp.min(jnp.where(noisy == m1, eidx, n_exp), axis=1, keepdims=True)
    masked = jnp.where(eidx == i1, -jnp.inf, noisy)
    m2 = jnp.max(masked, axis=1, keepdims=True)
    i2 = jnp.min(jnp.where(masked == m2, eidx, n_exp), axis=1, keepdims=True)
    on1 = eidx == i1
    on2 = eidx == i2
    member = jnp.logical_or(on1, on2)

    z = jnp.where(member, jnp.exp(noisy - m1), 0.0)
    probs = z / jnp.sum(z, axis=1, keepdims=True)

    m = member.astype(jnp.float32)
    inc = m
    d = 1
    while d < n_tok:
        shifted = jnp.concatenate(
            [jnp.zeros((d, n_exp), jnp.float32), inc[:-d, :]], axis=0)
        inc = inc + shifted
        d *= 2
    rank = inc - m
    keep = (rank < float(cap)).astype(jnp.float32)
    gates = probs * keep

    g0_ref[...] = jnp.sum(jnp.where(on1, gates, 0.0), axis=1, keepdims=True)
    g1_ref[...] = jnp.sum(jnp.where(on2, gates, 0.0), axis=1, keepdims=True)
    rk0 = jnp.sum(jnp.where(on1, rank, 0.0), axis=1, keepdims=True)
    rk1 = jnp.sum(jnp.where(on2, rank, 0.0), axis=1, keepdims=True)
    cp1 = float(cap_pad - 1)
    s0_ref[...] = i1 * cap_pad + jnp.minimum(rk0, cp1).astype(jnp.int32)
    s1_ref[...] = i2 * cap_pad + jnp.minimum(rk1, cp1).astype(jnp.int32)

    ep = jnp.mean(probs, axis=0, keepdims=True)
    ec = jnp.mean(m, axis=0, keepdims=True)
    loss_ref[...] = float(n_exp) * jnp.sum(ep * ec, axis=1, keepdims=True)


def _ffn_kernel(xa_ref, xb_ref, w1_ref, b1_ref, w2_ref, b2_ref,
                oa_ref, ob_ref, acc_ref, *, nf):
    f = pl.program_id(1)
    dh = oa_ref.shape[1]
    x = jnp.concatenate([xa_ref[...], xb_ref[...]], axis=1)
    h = jnp.dot(x, w1_ref[0].astype(jnp.bfloat16),
                preferred_element_type=jnp.float32)
    h = jnp.maximum(h + b1_ref[0], 0.0).astype(jnp.bfloat16)
    y = jnp.dot(h, w2_ref[0].astype(jnp.bfloat16),
                preferred_element_type=jnp.float32)
    y = y + jnp.where(f == 0, 1.0, 0.0) * b2_ref[0]

    if nf == 1:
        oa_ref[...] = y[:, :dh].astype(jnp.bfloat16)
        ob_ref[...] = y[:, dh:].astype(jnp.bfloat16)
    else:
        @pl.when(f == 0)
        def _():
            acc_ref[...] = y

        @pl.when(jnp.logical_and(f != 0, f != nf - 1))
        def _():
            acc_ref[...] = acc_ref[...] + y

        @pl.when(f == nf - 1)
        def _():
            tot = acc_ref[...] + y
            oa_ref[...] = tot[:, :dh].astype(jnp.bfloat16)
            ob_ref[...] = tot[:, dh:].astype(jnp.bfloat16)


def _combine_kernel(y0a_ref, y0b_ref, y1a_ref, y1b_ref, g0_ref, g1_ref,
                    out_ref):
    dh = y0a_ref.shape[1]
    g0 = g0_ref[...]
    g1 = g1_ref[...]
    out_ref[:, :dh] = (g0 * y0a_ref[...].astype(jnp.float32)
                       + g1 * y1a_ref[...].astype(jnp.float32))
    out_ref[:, dh:] = (g0 * y0b_ref[...].astype(jnp.float32)
                       + g1 * y1b_ref[...].astype(jnp.float32))


def _pack(a):
    r, c = a.shape
    return jax.lax.bitcast_convert_type(
        a.reshape(r, c // 2, 2), jnp.uint32)


def _unpack(a):
    r, c = a.shape
    return jax.lax.bitcast_convert_type(a, jnp.bfloat16).reshape(r, 2 * c)


def _sc_dispatch(xp, s0r, s1r, rows):
    n, d = xp.shape
    npan = 2
    dq = d // npan
    mesh = plsc.VectorSubcoreMesh(core_axis_name="core",
                                  subcore_axis_name="subcore")

    @functools.partial(
        pl.kernel,
        out_type=[jax.ShapeDtypeStruct((rows, dq), jnp.uint32)
                  for _ in range(npan)],
        mesh=mesh)
    def dispatch(x_hbm, s0_hbm, s1_hbm, *o_hbms):
        def make_body(o_hbm):
            def body(x_vmem, i0_vmem, i1_vmem):
                pltpu.sync_copy(x_vmem, o_hbm.at[i0_vmem.at[0]])
                pltpu.sync_copy(x_vmem, o_hbm.at[i1_vmem.at[0]])
            return body

        for q in range(npan):
            pltpu.emit_pipeline(
                make_body(o_hbms[q]),
                grid=(n // _SC_W,),
                in_specs=[
                    pl.BlockSpec((_SC_W, dq),
                                 functools.partial(
                                     lambda qq, i: (i, qq), q)),
                    pl.BlockSpec((1, _SC_W), lambda i: (0, i)),
                    pl.BlockSpec((1, _SC_W), lambda i: (0, i)),
                ],
                out_specs=[],
                core_axis_name=("core", "subcore"),
                dimension_semantics=(pltpu.PARALLEL,),
            )(x_hbm, s0_hbm, s1_hbm)

    return dispatch(xp, s0r, s1r)


def _sc_combine(yps, s0r, s1r):
    n = s0r.shape[1]
    dq = yps[0].shape[1]
    npan = len(yps)
    mesh = plsc.VectorSubcoreMesh(core_axis_name="core",
                                  subcore_axis_name="subcore")

    @functools.partial(
        pl.kernel,
        out_type=[jax.ShapeDtypeStruct((n, dq), jnp.uint32)
                  for _ in range(2 * npan)],
        mesh=mesh)
    def combine(*refs):
        y_hbms = refs[:npan]
        s_hbms = refs[npan:npan + 2]
        o_hbms = refs[npan + 2:]

        def make_body(y_hbm):
            def body(i_vmem, o_vmem):
                pltpu.sync_copy(y_hbm.at[i_vmem.at[0]], o_vmem)
            return body

        for k in range(2):
            for q in range(npan):
                pltpu.emit_pipeline(
                    make_body(y_hbms[q]),
                    grid=(n // _SC_W,),
                    in_specs=[pl.BlockSpec((1, _SC_W), lambda i: (0, i))],
                    out_specs=[pl.BlockSpec((_SC_W, dq),
                                            lambda i: (i, 0))],
                    core_axis_name=("core", "subcore"),
                    dimension_semantics=(pltpu.PARALLEL,),
                )(s_hbms[k], o_hbms[k * npan + q])

    return combine(*yps, s0r, s1r)


def kernel(x, Wg, bg, Wn, bn, W1, b1, W2, b2, noise):
    B, S, D = x.shape
    E = Wg.shape[1]
    F = W1.shape[2]
    N = B * S
    cap = int(N * _TOPK / E * _CAPF)
    cap_pad = ((cap + 127) // 128) * 128
    rows = E * cap_pad
    DH = D // 2

    xf = x.reshape(N, D)
    nz = noise.reshape(N, E)

    g0, g1, s0, s1, loss = pl.pallas_call(
        functools.partial(_router_kernel, cap=cap, cap_pad=cap_pad, n_exp=E),
        out_shape=[jax.ShapeDtypeStruct((N, 1), jnp.float32),
                   jax.ShapeDtypeStruct((N, 1), jnp.float32),
                   jax.ShapeDtypeStruct((N, 1), jnp.int32),
                   jax.ShapeDtypeStruct((N, 1), jnp.int32),
                   jax.ShapeDtypeStruct((1, 1), jnp.float32)],
    )(xf, Wg, bg.reshape(1, E), Wn, bn.reshape(1, E), nz)

    s0r = s0.reshape(1, N)
    s1r = s1.reshape(1, N)

    xp = _pack(xf.astype(jnp.bfloat16))
    xgp = _sc_dispatch(xp, s0r, s1r, rows)
    xga = _unpack(xgp[0])
    xgb = _unpack(xgp[1])

    FB = min(1024, F)
    NF = F // FB

    ya, yb = pl.pallas_call(
        functools.partial(_ffn_kernel, nf=NF),
        grid=(E, NF),
        in_specs=[
            pl.BlockSpec((cap_pad, DH), lambda e, f: (e, 0)),
            pl.BlockSpec((cap_pad, DH), lambda e, f: (e, 0)),
            pl.BlockSpec((1, D, FB), lambda e, f: (e, 0, f)),
            pl.BlockSpec((1, 1, FB), lambda e, f: (e, 0, f)),
            pl.BlockSpec((1, FB, D), lambda e, f: (e, f, 0)),
            pl.BlockSpec((1, 1, D), lambda e, f: (e, 0, 0)),
        ],
        out_specs=[
            pl.BlockSpec((cap_pad, DH), lambda e, f: (e, 0)),
            pl.BlockSpec((cap_pad, DH), lambda e, f: (e, 0)),
        ],
        out_shape=[jax.ShapeDtypeStruct((rows, DH), jnp.bfloat16),
                   jax.ShapeDtypeStruct((rows, DH), jnp.bfloat16)],
        scratch_shapes=[pltpu.VMEM((cap_pad, D), jnp.float32)],
        compiler_params=pltpu.CompilerParams(
            dimension_semantics=("arbitrary", "arbitrary")),
    )(xga, xgb, W1, b1.reshape(E, 1, F), W2, b2.reshape(E, 1, D))

    ypq = _sc_combine([_pack(ya), _pack(yb)], s0r, s1r)
    y0a, y0b, y1a, y1b = [_unpack(a) for a in ypq]

    BT = min(2048, N)
    out = pl.pallas_call(
        _combine_kernel,
        grid=(N // BT,),
        in_specs=[
            pl.BlockSpec((BT, DH), lambda t: (t, 0)),
            pl.BlockSpec((BT, DH), lambda t: (t, 0)),
            pl.BlockSpec((BT, DH), lambda t: (t, 0)),
            pl.BlockSpec((BT, DH), lambda t: (t, 0)),
            pl.BlockSpec((BT, 1), lambda t: (t, 0)),
            pl.BlockSpec((BT, 1), lambda t: (t, 0)),
        ],
        out_specs=pl.BlockSpec((BT, D), lambda t: (t, 0)),
        out_shape=jax.ShapeDtypeStruct((N, D), jnp.float32),
    )(y0a, y0b, y1a, y1b, g0, g1)

    return out.reshape(B, S, D), loss.reshape(())


# speedup vs baseline: 2.8938x; 2.8938x over previous
import functools

import jax
import jax.numpy as jnp
from jax.experimental import pallas as pl
from jax.experimental.pallas import tpu as pltpu
from jax.experimental.pallas import tpu_sc as plsc

_TOPK = 2
_CAPF = 1.1
_SC_W = 128
_NQ = 4


def _router_kernel(x_ref, wg_ref, bg_ref, wn_ref, bn_ref, nz_ref,
                   g0_ref, g1_ref, s0_ref, s1_ref, loss_ref,
                   *, cap, cap_pad, n_exp):
    x = x_ref[...].astype(jnp.bfloat16)
    lg = jnp.dot(x, wg_ref[...].astype(jnp.bfloat16),
                 preferred_element_type=jnp.float32) + bg_ref[...]
    ln = jnp.dot(x, wn_ref[...].astype(jnp.bfloat16),
                 preferred_element_type=jnp.float32) + bn_ref[...]
    sp = jnp.maximum(ln, 0.0) + jnp.log1p(jnp.exp(-jnp.abs(ln)))
    noisy = lg + nz_ref[...] * sp
    n_tok = noisy.shape[0]

    eidx = jax.lax.broadcasted_iota(jnp.int32, noisy.shape, 1)
    m1 = jnp.max(noisy, axis=1, keepdims=True)
    i1 = jnp.min(jnp.where(noisy == m1, eidx, n_exp), axis=1, keepdims=True)
    masked = jnp.where(eidx == i1, -jnp.inf, noisy)
    m2 = jnp.max(masked, axis=1, keepdims=True)
    i2 = jnp.min(jnp.where(masked == m2, eidx, n_exp), axis=1, keepdims=True)
    on1 = eidx == i1
    on2 = eidx == i2
    member = jnp.logical_or(on1, on2)

    z = jnp.where(member, jnp.exp(noisy - m1), 0.0)
    probs = z / jnp.sum(z, axis=1, keepdims=True)

    m = member.astype(jnp.float32)
    inc = m
    d = 1
    while d < n_tok:
        shifted = jnp.concatenate(
            [jnp.zeros((d, n_exp), jnp.float32), inc[:-d, :]], axis=0)
        inc = inc + shifted
        d *= 2
    rank = inc - m
    keep = (rank < float(cap)).astype(jnp.float32)
    gates = probs * keep

    g0_ref[...] = jnp.sum(jnp.where(on1, gates, 0.0), axis=1, keepdims=True)
    g1_ref[...] = jnp.sum(jnp.where(on2, gates, 0.0), axis=1, keepdims=True)
    rk0 = jnp.sum(jnp.where(on1, rank, 0.0), axis=1, keepdims=True)
    rk1 = jnp.sum(jnp.where(on2, rank, 0.0), axis=1, keepdims=True)
    cp1 = float(cap_pad - 1)
    s0_ref[...] = i1 * cap_pad + jnp.minimum(rk0, cp1).astype(jnp.int32)
    s1_ref[...] = i2 * cap_pad + jnp.minimum(rk1, cp1).astype(jnp.int32)

    ep = jnp.mean(probs, axis=0, keepdims=True)
    ec = jnp.mean(m, axis=0, keepdims=True)
    loss_ref[...] = float(n_exp) * jnp.sum(ep * ec, axis=1, keepdims=True)


def _ffn_kernel(*refs):
    xq_refs = refs[:_NQ]
    w1_ref, b1_ref, w2_ref, b2_ref = refs[_NQ:_NQ + 4]
    out_refs = refs[_NQ + 4:]
    f = pl.program_id(1)
    dq = xq_refs[0].shape[1]
    x = jnp.concatenate([r[...] for r in xq_refs],
                        axis=1).astype(jnp.bfloat16)
    h = jnp.dot(x, w1_ref[0].astype(jnp.bfloat16),
                preferred_element_type=jnp.float32)
    h = jnp.maximum(h + b1_ref[0], 0.0).astype(jnp.bfloat16)
    y = jnp.dot(h, w2_ref[0].astype(jnp.bfloat16),
                preferred_element_type=jnp.float32)
    y = y + jnp.where(f == 0, 1.0, 0.0) * b2_ref[0]

    @pl.when(f == 0)
    def _():
        for q, o_ref in enumerate(out_refs):
            o_ref[...] = y[:, q * dq:(q + 1) * dq]

    @pl.when(f != 0)
    def _():
        for q, o_ref in enumerate(out_refs):
            o_ref[...] = o_ref[...] + y[:, q * dq:(q + 1) * dq]


def _combine_kernel(*refs):
    yq0 = refs[:_NQ]
    yq1 = refs[_NQ:2 * _NQ]
    g0_ref, g1_ref, out_ref = refs[2 * _NQ:]
    dq = yq0[0].shape[1]
    g0 = g0_ref[...]
    g1 = g1_ref[...]
    for q in range(_NQ):
        out_ref[:, q * dq:(q + 1) * dq] = (g0 * yq0[q][...]
                                           + g1 * yq1[q][...])


def _sc_dispatch(xf, s0r, s1r, rows):
    n, d = xf.shape
    dq = d // _NQ
    mesh = plsc.VectorSubcoreMesh(core_axis_name="core",
                                  subcore_axis_name="subcore")

    @functools.partial(
        pl.kernel,
        out_type=[jax.ShapeDtypeStruct((rows, dq), jnp.float32)
                  for _ in range(_NQ)],
        mesh=mesh)
    def dispatch(x_hbm, s0_hbm, s1_hbm, *o_hbms):
        def make_body(o_hbm):
            def body(x_vmem, i0_vmem, i1_vmem):
                pltpu.sync_copy(x_vmem, o_hbm.at[i0_vmem.at[0]])
                pltpu.sync_copy(x_vmem, o_hbm.at[i1_vmem.at[0]])
            return body

        for q in range(_NQ):
            pltpu.emit_pipeline(
                make_body(o_hbms[q]),
                grid=(n // _SC_W,),
                in_specs=[
                    pl.BlockSpec((_SC_W, dq),
                                 functools.partial(
                                     lambda qq, i: (i, qq), q)),
                    pl.BlockSpec((1, _SC_W), lambda i: (0, i)),
                    pl.BlockSpec((1, _SC_W), lambda i: (0, i)),
                ],
                out_specs=[],
                core_axis_name=("core", "subcore"),
                dimension_semantics=(pltpu.PARALLEL,),
            )(x_hbm, s0_hbm, s1_hbm)

    return dispatch(xf, s0r, s1r)


def _sc_combine(yqs, s0r, s1r):
    n = s0r.shape[1]
    dq = yqs[0].shape[1]
    mesh = plsc.VectorSubcoreMesh(core_axis_name="core",
                                  subcore_axis_name="subcore")

    @functools.partial(
        pl.kernel,
        out_type=[jax.ShapeDtypeStruct((n, dq), jnp.float32)
                  for _ in range(2 * _NQ)],
        mesh=mesh)
    def combine(*refs):
        y_hbms = refs[:_NQ]
        s_hbms = refs[_NQ:_NQ + 2]
        o_hbms = refs[_NQ + 2:]

        def make_body(y_hbm):
            def body(i_vmem, o_vmem):
                pltpu.sync_copy(y_hbm.at[i_vmem.at[0]], o_vmem)
            return body

        for k in range(2):
            for q in range(_NQ):
                pltpu.emit_pipeline(
                    make_body(y_hbms[q]),
                    grid=(n // _SC_W,),
                    in_specs=[pl.BlockSpec((1, _SC_W), lambda i: (0, i))],
                    out_specs=[pl.BlockSpec((_SC_W, dq),
                                            lambda i: (i, 0))],
                    core_axis_name=("core", "subcore"),
                    dimension_semantics=(pltpu.PARALLEL,),
                )(s_hbms[k], o_hbms[k * _NQ + q])

    return combine(*yqs, s0r, s1r)


def kernel(x, Wg, bg, Wn, bn, W1, b1, W2, b2, noise):
    B, S, D = x.shape
    E = Wg.shape[1]
    F = W1.shape[2]
    N = B * S
    cap = int(N * _TOPK / E * _CAPF)
    cap_pad = ((cap + 127) // 128) * 128
    rows = E * cap_pad
    DQ = D // _NQ

    xf = x.reshape(N, D)
    nz = noise.reshape(N, E)

    g0, g1, s0, s1, loss = pl.pallas_call(
        functools.partial(_router_kernel, cap=cap, cap_pad=cap_pad, n_exp=E),
        out_shape=[jax.ShapeDtypeStruct((N, 1), jnp.float32),
                   jax.ShapeDtypeStruct((N, 1), jnp.float32),
                   jax.ShapeDtypeStruct((N, 1), jnp.int32),
                   jax.ShapeDtypeStruct((N, 1), jnp.int32),
                   jax.ShapeDtypeStruct((1, 1), jnp.float32)],
    )(xf, Wg, bg.reshape(1, E), Wn, bn.reshape(1, E), nz)

    s0r = s0.reshape(1, N)
    s1r = s1.reshape(1, N)

    xgq = _sc_dispatch(xf, s0r, s1r, rows)

    FB = min(1024, F)
    NF = F // FB

    yq = pl.pallas_call(
        _ffn_kernel,
        grid=(E, NF),
        in_specs=(
            [pl.BlockSpec((cap_pad, DQ), lambda e, f: (e, 0))
             for _ in range(_NQ)]
            + [
                pl.BlockSpec((1, D, FB), lambda e, f: (e, 0, f)),
                pl.BlockSpec((1, 1, FB), lambda e, f: (e, 0, f)),
                pl.BlockSpec((1, FB, D), lambda e, f: (e, f, 0)),
                pl.BlockSpec((1, 1, D), lambda e, f: (e, 0, 0)),
            ]),
        out_specs=[pl.BlockSpec((cap_pad, DQ), lambda e, f: (e, 0))
                   for _ in range(_NQ)],
        out_shape=[jax.ShapeDtypeStruct((rows, DQ), jnp.float32)
                   for _ in range(_NQ)],
        compiler_params=pltpu.CompilerParams(
            dimension_semantics=("arbitrary", "arbitrary")),
    )(*xgq, W1, b1.reshape(E, 1, F), W2, b2.reshape(E, 1, D))

    ypq = _sc_combine(yq, s0r, s1r)

    BT = min(2048, N)
    out = pl.pallas_call(
        _combine_kernel,
        grid=(N // BT,),
        in_specs=(
            [pl.BlockSpec((BT, DQ), lambda t: (t, 0))
             for _ in range(2 * _NQ)]
            + [pl.BlockSpec((BT, 1), lambda t: (t, 0)),
               pl.BlockSpec((BT, 1), lambda t: (t, 0))]),
        out_specs=pl.BlockSpec((BT, D), lambda t: (t, 0)),
        out_shape=jax.ShapeDtypeStruct((N, D), jnp.float32),
    )(*ypq, g0, g1)

    return out.reshape(B, S, D), loss.reshape(())


# speedup vs baseline: 2.9880x; 1.0326x over previous
import functools

import jax
import jax.numpy as jnp
from jax.experimental import pallas as pl
from jax.experimental.pallas import tpu as pltpu
from jax.experimental.pallas import tpu_sc as plsc

_TOPK = 2
_CAPF = 1.1
_SC_W = 128
_NQ = 4


def _router_kernel(x_ref, wg_ref, bg_ref, wn_ref, bn_ref, nz_ref,
                   g0_ref, g1_ref, s0_ref, s1_ref, loss_ref,
                   *, cap, cap_pad, n_exp):
    x = x_ref[...].astype(jnp.bfloat16)
    lg = jnp.dot(x, wg_ref[...].astype(jnp.bfloat16),
                 preferred_element_type=jnp.float32) + bg_ref[...]
    ln = jnp.dot(x, wn_ref[...].astype(jnp.bfloat16),
                 preferred_element_type=jnp.float32) + bn_ref[...]
    sp = jnp.maximum(ln, 0.0) + jnp.log1p(jnp.exp(-jnp.abs(ln)))
    noisy = lg + nz_ref[...] * sp
    n_tok = noisy.shape[0]

    eidx = jax.lax.broadcasted_iota(jnp.int32, noisy.shape, 1)
    m1 = jnp.max(noisy, axis=1, keepdims=True)
    i1 = jnp.min(jnp.where(noisy == m1, eidx, n_exp), axis=1, keepdims=True)
    masked = jnp.where(eidx == i1, -jnp.inf, noisy)
    m2 = jnp.max(masked, axis=1, keepdims=True)
    i2 = jnp.min(jnp.where(masked == m2, eidx, n_exp), axis=1, keepdims=True)
    on1 = eidx == i1
    on2 = eidx == i2
    member = jnp.logical_or(on1, on2)

    z = jnp.where(member, jnp.exp(noisy - m1), 0.0)
    probs = z / jnp.sum(z, axis=1, keepdims=True)

    m = member.astype(jnp.float32)
    inc = m
    d = 1
    while d < n_tok:
        shifted = jnp.concatenate(
            [jnp.zeros((d, n_exp), jnp.float32), inc[:-d, :]], axis=0)
        inc = inc + shifted
        d *= 2
    rank = inc - m
    keep = (rank < float(cap)).astype(jnp.float32)
    gates = probs * keep

    g0_ref[...] = jnp.sum(jnp.where(on1, gates, 0.0), axis=1, keepdims=True)
    g1_ref[...] = jnp.sum(jnp.where(on2, gates, 0.0), axis=1, keepdims=True)
    rk0 = jnp.sum(jnp.where(on1, rank, 0.0), axis=1, keepdims=True)
    rk1 = jnp.sum(jnp.where(on2, rank, 0.0), axis=1, keepdims=True)
    cp1 = float(cap_pad - 1)
    s0_ref[...] = i1 * cap_pad + jnp.minimum(rk0, cp1).astype(jnp.int32)
    s1_ref[...] = i2 * cap_pad + jnp.minimum(rk1, cp1).astype(jnp.int32)

    ep = jnp.mean(probs, axis=0, keepdims=True)
    ec = jnp.mean(m, axis=0, keepdims=True)
    loss_ref[...] = float(n_exp) * jnp.sum(ep * ec, axis=1, keepdims=True)


def _ffn_kernel(*refs):
    xq_refs = refs[:_NQ]
    w1_ref, b1_ref, w2_ref, b2_ref = refs[_NQ:_NQ + 4]
    out_refs = refs[_NQ + 4:]
    f = pl.program_id(1)
    dq = xq_refs[0].shape[1]
    x = jnp.concatenate([r[...] for r in xq_refs],
                        axis=1).astype(jnp.bfloat16)
    h = jnp.dot(x, w1_ref[0].astype(jnp.bfloat16),
                preferred_element_type=jnp.float32)
    h = jnp.maximum(h + b1_ref[0], 0.0).astype(jnp.bfloat16)
    y = jnp.dot(h, w2_ref[0].astype(jnp.bfloat16),
                preferred_element_type=jnp.float32)
    y = y + jnp.where(f == 0, 1.0, 0.0) * b2_ref[0]

    @pl.when(f == 0)
    def _():
        for q, o_ref in enumerate(out_refs):
            o_ref[...] = y[:, q * dq:(q + 1) * dq]

    @pl.when(f != 0)
    def _():
        for q, o_ref in enumerate(out_refs):
            o_ref[...] = o_ref[...] + y[:, q * dq:(q + 1) * dq]


def _combine_kernel(*refs):
    yq0 = refs[:_NQ]
    yq1 = refs[_NQ:2 * _NQ]
    g0_ref, g1_ref, out_ref = refs[2 * _NQ:]
    dq = yq0[0].shape[1]
    g0 = g0_ref[...]
    g1 = g1_ref[...]
    for q in range(_NQ):
        out_ref[:, q * dq:(q + 1) * dq] = (g0 * yq0[q][...]
                                           + g1 * yq1[q][...])


def _sc_dispatch(xf, s0r, s1r, rows):
    n, d = xf.shape
    dq = d // _NQ
    mesh = plsc.VectorSubcoreMesh(core_axis_name="core",
                                  subcore_axis_name="subcore")

    @functools.partial(
        pl.kernel,
        out_type=[jax.ShapeDtypeStruct((rows, dq), jnp.float32)
                  for _ in range(_NQ)],
        mesh=mesh)
    def dispatch(x_hbm, s0_hbm, s1_hbm, *o_hbms):
        def make_body(o_hbm):
            def body(x_vmem, i0_vmem, i1_vmem):
                pltpu.sync_copy(x_vmem, o_hbm.at[i0_vmem.at[0]])
                pltpu.sync_copy(x_vmem, o_hbm.at[i1_vmem.at[0]])
            return body

        for q in range(_NQ):
            pltpu.emit_pipeline(
                make_body(o_hbms[q]),
                grid=(n // _SC_W,),
                in_specs=[
                    pl.BlockSpec((_SC_W, dq),
                                 functools.partial(
                                     lambda qq, i: (i, qq), q)),
                    pl.BlockSpec((1, _SC_W), lambda i: (0, i)),
                    pl.BlockSpec((1, _SC_W), lambda i: (0, i)),
                ],
                out_specs=[],
                core_axis_name=("core", "subcore"),
                dimension_semantics=(pltpu.PARALLEL,),
            )(x_hbm, s0_hbm, s1_hbm)

    return dispatch(xf, s0r, s1r)


def _sc_combine(yqs, s0r, s1r):
    n = s0r.shape[1]
    dq = yqs[0].shape[1]
    mesh = plsc.VectorSubcoreMesh(core_axis_name="core",
                                  subcore_axis_name="subcore")

    @functools.partial(
        pl.kernel,
        out_type=[jax.ShapeDtypeStruct((n, dq), jnp.float32)
                  for _ in range(2 * _NQ)],
        mesh=mesh)
    def combine(*refs):
        y_hbms = refs[:_NQ]
        s_hbms = refs[_NQ:_NQ + 2]
        o_hbms = refs[_NQ + 2:]

        def make_body(y_hbm):
            def body(i_vmem, o_vmem):
                pltpu.sync_copy(y_hbm.at[i_vmem.at[0]], o_vmem)
            return body

        for k in range(2):
            for q in range(_NQ):
                pltpu.emit_pipeline(
                    make_body(y_hbms[q]),
                    grid=(n // _SC_W,),
                    in_specs=[pl.BlockSpec((1, _SC_W), lambda i: (0, i))],
                    out_specs=[pl.BlockSpec((_SC_W, dq),
                                            lambda i: (i, 0))],
                    core_axis_name=("core", "subcore"),
                    dimension_semantics=(pltpu.PARALLEL,),
                )(s_hbms[k], o_hbms[k * _NQ + q])

    return combine(*yqs, s0r, s1r)


def kernel(x, Wg, bg, Wn, bn, W1, b1, W2, b2, noise):
    B, S, D = x.shape
    E = Wg.shape[1]
    F = W1.shape[2]
    N = B * S
    cap = int(N * _TOPK / E * _CAPF)
    cap_pad = ((cap + 127) // 128) * 128
    rows = E * cap_pad
    DQ = D // _NQ

    xf = x.reshape(N, D)
    nz = noise.reshape(N, E)

    g0, g1, s0, s1, loss = pl.pallas_call(
        functools.partial(_router_kernel, cap=cap, cap_pad=cap_pad, n_exp=E),
        out_shape=[jax.ShapeDtypeStruct((N, 1), jnp.float32),
                   jax.ShapeDtypeStruct((N, 1), jnp.float32),
                   jax.ShapeDtypeStruct((N, 1), jnp.int32),
                   jax.ShapeDtypeStruct((N, 1), jnp.int32),
                   jax.ShapeDtypeStruct((1, 1), jnp.float32)],
    )(xf, Wg, bg.reshape(1, E), Wn, bn.reshape(1, E), nz)

    s0r = s0.reshape(1, N)
    s1r = s1.reshape(1, N)

    xgq = _sc_dispatch(xf, s0r, s1r, rows)

    FB = min(2048, F)
    NF = F // FB

    yq = pl.pallas_call(
        _ffn_kernel,
        grid=(E, NF),
        in_specs=(
            [pl.BlockSpec((cap_pad, DQ), lambda e, f: (e, 0))
             for _ in range(_NQ)]
            + [
                pl.BlockSpec((1, D, FB), lambda e, f: (e, 0, f)),
                pl.BlockSpec((1, 1, FB), lambda e, f: (e, 0, f)),
                pl.BlockSpec((1, FB, D), lambda e, f: (e, f, 0)),
                pl.BlockSpec((1, 1, D), lambda e, f: (e, 0, 0)),
            ]),
        out_specs=[pl.BlockSpec((cap_pad, DQ), lambda e, f: (e, 0))
                   for _ in range(_NQ)],
        out_shape=[jax.ShapeDtypeStruct((rows, DQ), jnp.float32)
                   for _ in range(_NQ)],
        compiler_params=pltpu.CompilerParams(
            dimension_semantics=("arbitrary", "arbitrary")),
    )(*xgq, W1, b1.reshape(E, 1, F), W2, b2.reshape(E, 1, D))

    ypq = _sc_combine(yq, s0r, s1r)

    BT = min(2048, N)
    out = pl.pallas_call(
        _combine_kernel,
        grid=(N // BT,),
        in_specs=(
            [pl.BlockSpec((BT, DQ), lambda t: (t, 0))
             for _ in range(2 * _NQ)]
            + [pl.BlockSpec((BT, 1), lambda t: (t, 0)),
               pl.BlockSpec((BT, 1), lambda t: (t, 0))]),
        out_specs=pl.BlockSpec((BT, D), lambda t: (t, 0)),
        out_shape=jax.ShapeDtypeStruct((N, D), jnp.float32),
    )(*ypq, g0, g1)

    return out.reshape(B, S, D), loss.reshape(())


# speedup vs baseline: 3.0115x; 1.0079x over previous
import functools

import jax
import jax.numpy as jnp
from jax.experimental import pallas as pl
from jax.experimental.pallas import tpu as pltpu
from jax.experimental.pallas import tpu_sc as plsc

_TOPK = 2
_CAPF = 1.1
_SC_W = 128
_NQ = 4


def _router_kernel(x_ref, wg_ref, bg_ref, wn_ref, bn_ref, nz_ref,
                   g0_ref, g1_ref, s0_ref, s1_ref, loss_ref,
                   *, cap, cap_pad, n_exp):
    x = x_ref[...].astype(jnp.bfloat16)
    lg = jnp.dot(x, wg_ref[...].astype(jnp.bfloat16),
                 preferred_element_type=jnp.float32) + bg_ref[...]
    ln = jnp.dot(x, wn_ref[...].astype(jnp.bfloat16),
                 preferred_element_type=jnp.float32) + bn_ref[...]
    sp = jnp.maximum(ln, 0.0) + jnp.log1p(jnp.exp(-jnp.abs(ln)))
    noisy = lg + nz_ref[...] * sp
    n_tok = noisy.shape[0]

    eidx = jax.lax.broadcasted_iota(jnp.int32, noisy.shape, 1)
    m1 = jnp.max(noisy, axis=1, keepdims=True)
    i1 = jnp.min(jnp.where(noisy == m1, eidx, n_exp), axis=1, keepdims=True)
    masked = jnp.where(eidx == i1, -jnp.inf, noisy)
    m2 = jnp.max(masked, axis=1, keepdims=True)
    i2 = jnp.min(jnp.where(masked == m2, eidx, n_exp), axis=1, keepdims=True)
    on1 = eidx == i1
    on2 = eidx == i2
    member = jnp.logical_or(on1, on2)

    z = jnp.where(member, jnp.exp(noisy - m1), 0.0)
    probs = z / jnp.sum(z, axis=1, keepdims=True)

    m = member.astype(jnp.float32)
    inc = m
    d = 1
    while d < n_tok:
        shifted = jnp.concatenate(
            [jnp.zeros((d, n_exp), jnp.float32), inc[:-d, :]], axis=0)
        inc = inc + shifted
        d *= 2
    rank = inc - m
    keep = (rank < float(cap)).astype(jnp.float32)
    gates = probs * keep

    g0_ref[...] = jnp.sum(jnp.where(on1, gates, 0.0), axis=1, keepdims=True)
    g1_ref[...] = jnp.sum(jnp.where(on2, gates, 0.0), axis=1, keepdims=True)
    rk0 = jnp.sum(jnp.where(on1, rank, 0.0), axis=1, keepdims=True)
    rk1 = jnp.sum(jnp.where(on2, rank, 0.0), axis=1, keepdims=True)
    cp1 = float(cap_pad - 1)
    s0_ref[...] = i1 * cap_pad + jnp.minimum(rk0, cp1).astype(jnp.int32)
    s1_ref[...] = i2 * cap_pad + jnp.minimum(rk1, cp1).astype(jnp.int32)

    ep = jnp.mean(probs, axis=0, keepdims=True)
    ec = jnp.mean(m, axis=0, keepdims=True)
    loss_ref[...] = float(n_exp) * jnp.sum(ep * ec, axis=1, keepdims=True)


def _ffn_kernel(*refs):
    xq_refs = refs[:_NQ]
    w1_ref, b1_ref, w2_ref, b2_ref = refs[_NQ:_NQ + 4]
    out_refs = refs[_NQ + 4:]
    f = pl.program_id(1)
    dq = xq_refs[0].shape[1]
    x = jnp.concatenate([r[...] for r in xq_refs],
                        axis=1).astype(jnp.bfloat16)
    h = jnp.dot(x, w1_ref[0].astype(jnp.bfloat16),
                preferred_element_type=jnp.float32)
    h = jnp.maximum(h + b1_ref[0], 0.0).astype(jnp.bfloat16)
    y = jnp.dot(h, w2_ref[0].astype(jnp.bfloat16),
                preferred_element_type=jnp.float32)
    y = y + jnp.where(f == 0, 1.0, 0.0) * b2_ref[0]

    @pl.when(f == 0)
    def _():
        for q, o_ref in enumerate(out_refs):
            o_ref[...] = y[:, q * dq:(q + 1) * dq]

    @pl.when(f != 0)
    def _():
        for q, o_ref in enumerate(out_refs):
            o_ref[...] = o_ref[...] + y[:, q * dq:(q + 1) * dq]


def _combine_kernel(*refs):
    yq0 = refs[:_NQ]
    yq1 = refs[_NQ:2 * _NQ]
    g0_ref, g1_ref, out_ref = refs[2 * _NQ:]
    dq = yq0[0].shape[1]
    g0 = g0_ref[...]
    g1 = g1_ref[...]
    for q in range(_NQ):
        out_ref[:, q * dq:(q + 1) * dq] = (g0 * yq0[q][...]
                                           + g1 * yq1[q][...])


def _sc_dispatch(xf, s0r, s1r, rows):
    n, d = xf.shape
    dq = d // _NQ
    mesh = plsc.VectorSubcoreMesh(core_axis_name="core",
                                  subcore_axis_name="subcore")

    @functools.partial(
        pl.kernel,
        out_type=[jax.ShapeDtypeStruct((rows, dq), jnp.float32)
                  for _ in range(_NQ)],
        mesh=mesh)
    def dispatch(x_hbm, s0_hbm, s1_hbm, *o_hbms):
        def make_body(o_hbm):
            def body(x_vmem, i0_vmem, i1_vmem):
                pltpu.sync_copy(x_vmem, o_hbm.at[i0_vmem.at[0]])
                pltpu.sync_copy(x_vmem, o_hbm.at[i1_vmem.at[0]])
            return body

        for q in range(_NQ):
            pltpu.emit_pipeline(
                make_body(o_hbms[q]),
                grid=(n // _SC_W,),
                in_specs=[
                    pl.BlockSpec((_SC_W, dq),
                                 functools.partial(
                                     lambda qq, i: (i, qq), q)),
                    pl.BlockSpec((1, _SC_W), lambda i: (0, i)),
                    pl.BlockSpec((1, _SC_W), lambda i: (0, i)),
                ],
                out_specs=[],
                core_axis_name=("core", "subcore"),
                dimension_semantics=(pltpu.PARALLEL,),
            )(x_hbm, s0_hbm, s1_hbm)

    return dispatch(xf, s0r, s1r)


def _sc_combine(yqs, s0r, s1r):
    n = s0r.shape[1]
    dq = yqs[0].shape[1]
    mesh = plsc.VectorSubcoreMesh(core_axis_name="core",
                                  subcore_axis_name="subcore")

    @functools.partial(
        pl.kernel,
        out_type=[jax.ShapeDtypeStruct((n, dq), jnp.float32)
                  for _ in range(2 * _NQ)],
        mesh=mesh)
    def combine(*refs):
        y_hbms = refs[:_NQ]
        s_hbms = refs[_NQ:_NQ + 2]
        o_hbms = refs[_NQ + 2:]

        def make_body(y_hbm):
            def body(i_vmem, o_vmem):
                pltpu.sync_copy(y_hbm.at[i_vmem.at[0]], o_vmem)
            return body

        for k in range(2):
            for q in range(_NQ):
                pltpu.emit_pipeline(
                    make_body(y_hbms[q]),
                    grid=(n // _SC_W,),
                    in_specs=[pl.BlockSpec((1, _SC_W), lambda i: (0, i))],
                    out_specs=[pl.BlockSpec((_SC_W, dq),
                                            lambda i: (i, 0))],
                    core_axis_name=("core", "subcore"),
                    dimension_semantics=(pltpu.PARALLEL,),
                )(s_hbms[k], o_hbms[k * _NQ + q])

    return combine(*yqs, s0r, s1r)


def kernel(x, Wg, bg, Wn, bn, W1, b1, W2, b2, noise):
    B, S, D = x.shape
    E = Wg.shape[1]
    F = W1.shape[2]
    N = B * S
    cap = int(N * _TOPK / E * _CAPF)
    cap_pad = ((cap + 7) // 8) * 8
    rows = E * cap_pad
    DQ = D // _NQ

    xf = x.reshape(N, D)
    nz = noise.reshape(N, E)

    g0, g1, s0, s1, loss = pl.pallas_call(
        functools.partial(_router_kernel, cap=cap, cap_pad=cap_pad, n_exp=E),
        out_shape=[jax.ShapeDtypeStruct((N, 1), jnp.float32),
                   jax.ShapeDtypeStruct((N, 1), jnp.float32),
                   jax.ShapeDtypeStruct((N, 1), jnp.int32),
                   jax.ShapeDtypeStruct((N, 1), jnp.int32),
                   jax.ShapeDtypeStruct((1, 1), jnp.float32)],
    )(xf, Wg, bg.reshape(1, E), Wn, bn.reshape(1, E), nz)

    s0r = s0.reshape(1, N)
    s1r = s1.reshape(1, N)

    xgq = _sc_dispatch(xf, s0r, s1r, rows)

    FB = min(2048, F)
    NF = F // FB

    yq = pl.pallas_call(
        _ffn_kernel,
        grid=(E, NF),
        in_specs=(
            [pl.BlockSpec((cap_pad, DQ), lambda e, f: (e, 0))
             for _ in range(_NQ)]
            + [
                pl.BlockSpec((1, D, FB), lambda e, f: (e, 0, f)),
                pl.BlockSpec((1, 1, FB), lambda e, f: (e, 0, f)),
                pl.BlockSpec((1, FB, D), lambda e, f: (e, f, 0)),
                pl.BlockSpec((1, 1, D), lambda e, f: (e, 0, 0)),
            ]),
        out_specs=[pl.BlockSpec((cap_pad, DQ), lambda e, f: (e, 0))
                   for _ in range(_NQ)],
        out_shape=[jax.ShapeDtypeStruct((rows, DQ), jnp.float32)
                   for _ in range(_NQ)],
        compiler_params=pltpu.CompilerParams(
            dimension_semantics=("arbitrary", "arbitrary")),
    )(*xgq, W1, b1.reshape(E, 1, F), W2, b2.reshape(E, 1, D))

    ypq = _sc_combine(yq, s0r, s1r)

    BT = min(2048, N)
    out = pl.pallas_call(
        _combine_kernel,
        grid=(N // BT,),
        in_specs=(
            [pl.BlockSpec((BT, DQ), lambda t: (t, 0))
             for _ in range(2 * _NQ)]
            + [pl.BlockSpec((BT, 1), lambda t: (t, 0)),
               pl.BlockSpec((BT, 1), lambda t: (t, 0))]),
        out_specs=pl.BlockSpec((BT, D), lambda t: (t, 0)),
        out_shape=jax.ShapeDtypeStruct((N, D), jnp.float32),
    )(*ypq, g0, g1)

    return out.reshape(B, S, D), loss.reshape(())
